# Initial kernel scaffold; baseline (speedup 1.0000x reference)
import jax
import jax.numpy as jnp
from jax.experimental import pallas as pl


def kernel(x, mlp_user_w, mlp_item_w, gmf_user_w, gmf_item_w, W1, b1, W2, b2, W3, b3, W_last, b_last):
    raise NotImplementedError("write your pallas kernel here")


# speedup vs baseline: 2.5739x; 2.5739x over previous
import functools

import jax
import jax.numpy as jnp
from jax import lax
from jax.experimental import pallas as pl
from jax.experimental.pallas import tpu as pltpu
from jax.experimental.pallas import tpu_sc as plsc

NUM_USER = 100000
NUM_ITEM = 100000
DIM = 128
BATCH = 16384

_info = plsc.get_sparse_core_info()
NC, NS, L = _info.num_cores, _info.num_subcores, _info.num_lanes
NW = NC * NS
BPW = BATCH // NW
NCHUNK = BPW // 128

_sc_mesh = plsc.VectorSubcoreMesh(core_axis_name="c", subcore_axis_name="s")


@functools.partial(
    pl.kernel,
    mesh=_sc_mesh,
    out_type=[jax.ShapeDtypeStruct((BATCH, DIM), jnp.float32) for _ in range(4)],
    scratch_types=[
        pltpu.VMEM((NCHUNK, 128), jnp.int32),
        pltpu.VMEM((NCHUNK, 128), jnp.int32),
        pltpu.VMEM((BPW, DIM), jnp.float32),
        pltpu.SemaphoreType.DMA,
    ],
)
def _sc_gather(uidx_hbm, iidx_hbm, tab_mu, tab_mi, tab_gu, tab_gi,
               out_mu, out_mi, out_gu, out_gi, idx_u, idx_i, rows, sem):
    wid = lax.axis_index("s") * NC + lax.axis_index("c")
    base = wid * BPW
    pltpu.sync_copy(uidx_hbm.at[wid], idx_u)
    pltpu.sync_copy(iidx_hbm.at[wid], idx_i)
    for table, idx, out in (
        (tab_mu, idx_u, out_mu),
        (tab_mi, idx_i, out_mi),
        (tab_gu, idx_u, out_gu),
        (tab_gi, idx_i, out_gi),
    ):
        handles = []
        for c in range(NCHUNK):
            handles.append(
                pltpu.async_copy(table.at[idx.at[c]],
                                 rows.at[pl.ds(c * 128, 128)], sem))
        for h in handles:
            h.wait()
        pltpu.sync_copy(rows, out.at[pl.ds(base, BPW)])


_TC_BLK = 2048


def _tc_body(mu_ref, mi_ref, gu_ref, gi_ref, w1a_ref, w1b_ref, b1_ref,
             w2_ref, b2_ref, w3_ref, b3_ref, wg_ref, wm_ref, bl_ref, out_ref):
    f32 = jnp.float32
    h = jnp.dot(mu_ref[...], w1a_ref[...], preferred_element_type=f32)
    h += jnp.dot(mi_ref[...], w1b_ref[...], preferred_element_type=f32)
    h = jnp.maximum(h + b1_ref[...], 0.0)
    h = jnp.maximum(jnp.dot(h, w2_ref[...], preferred_element_type=f32)
                    + b2_ref[...], 0.0)
    h = jnp.maximum(jnp.dot(h, w3_ref[...], preferred_element_type=f32)
                    + b3_ref[...], 0.0)
    g = gu_ref[...] * gi_ref[...]
    out = jnp.dot(g, wg_ref[...], preferred_element_type=f32)
    out += jnp.dot(h, wm_ref[...], preferred_element_type=f32)
    out_ref[...] = out + bl_ref[...]


def _fixed(shape):
    return pl.BlockSpec(shape, lambda b: (0, 0))


_tc_dense = pl.pallas_call(
    _tc_body,
    grid=(BATCH // _TC_BLK,),
    in_specs=[
        pl.BlockSpec((_TC_BLK, DIM), lambda b: (b, 0)),
        pl.BlockSpec((_TC_BLK, DIM), lambda b: (b, 0)),
        pl.BlockSpec((_TC_BLK, DIM), lambda b: (b, 0)),
        pl.BlockSpec((_TC_BLK, DIM), lambda b: (b, 0)),
        _fixed((DIM, 64)),
        _fixed((DIM, 64)),
        _fixed((1, 64)),
        _fixed((64, 16)),
        _fixed((1, 16)),
        _fixed((16, 8)),
        _fixed((1, 8)),
        _fixed((DIM, 1)),
        _fixed((8, 1)),
        _fixed((1, 1)),
    ],
    out_specs=pl.BlockSpec((_TC_BLK, 1), lambda b: (b, 0)),
    out_shape=jax.ShapeDtypeStruct((BATCH, 1), jnp.float32),
)


def kernel(x, mlp_user_w, mlp_item_w, gmf_user_w, gmf_item_w,
           W1, b1, W2, b2, W3, b3, W_last, b_last):
    u = x[:, 0].astype(jnp.int32).reshape(NW, NCHUNK, 128)
    i = x[:, 1].astype(jnp.int32).reshape(NW, NCHUNK, 128)
    mu, mi, gu, gi = _sc_gather(u, i, mlp_user_w, mlp_item_w,
                                gmf_user_w, gmf_item_w)
    w1a = W1[:, :DIM].T
    w1b = W1[:, DIM:].T
    wg = W_last[0, :DIM].reshape(DIM, 1)
    wm = W_last[0, DIM:].reshape(8, 1)
    out = _tc_dense(mu, mi, gu, gi, w1a, w1b, b1.reshape(1, 64),
                    W2.T, b2.reshape(1, 16), W3.T, b3.reshape(1, 8),
                    wg, wm, b_last.reshape(1, 1))
    return out
